# Initial kernel scaffold; baseline (speedup 1.0000x reference)
import jax
import jax.numpy as jnp
from jax.experimental import pallas as pl


def kernel(X, r_w, W_qkv, W_o):
    raise NotImplementedError("write your pallas kernel here")


# speedup vs baseline: 1.2876x; 1.2876x over previous
import functools

import jax
import jax.numpy as jnp
from jax import lax
from jax.experimental import pallas as pl
from jax.experimental.pallas import tpu as pltpu
from jax.experimental.pallas import tpu_sc as plsc

B, T, H = 2, 8192, 768
E, HP = 12, 64
K = 256
N_ROT = 32
ROPE_BASE = 10000.0
BE = B * E
EK = E * K
NC, NS, L = 2, 16, 16
NW = NC * NS
MINI32 = -2147483648

@functools.lru_cache(maxsize=None)
def _sc_mesh():
    return plsc.VectorSubcoreMesh(
        core_axis_name="c", subcore_axis_name="s", num_cores=NC, num_subcores=NS)


def _f32_key(v):
    i = lax.bitcast_convert_type(v, jnp.int32)
    return jnp.where(i >= 0, i, jnp.bitwise_xor(jnp.bitwise_not(i), jnp.int32(MINI32)))


_TCH = 2048


def _router_body(x_ref, w_ref, lg_ref):
    x = x_ref[0]
    w = w_ref[...]
    lg_ref[0] = lax.dot_general(w, x, (((1,), (1,)), ((), ())),
                                preferred_element_type=jnp.float32)


def _router(X, r_w):
    return pl.pallas_call(
        _router_body,
        grid=(B, T // _TCH),
        in_specs=[
            pl.BlockSpec((1, _TCH, H), lambda b, g: (b, g, 0)),
            pl.BlockSpec((E, H), lambda b, g: (0, 0)),
        ],
        out_specs=pl.BlockSpec((1, E, _TCH), lambda b, g: (b, 0, g)),
        out_shape=jax.ShapeDtypeStruct((B, E, T), jnp.float32),
    )(X, r_w)


def _thresh_body(lg_ref, thr_ref):
    keys = _f32_key(lg_ref[...])

    def step(_, carry):
        lo, hi = carry
        mid = (lo >> 1) + (hi >> 1) + (lo & hi & 1)
        cnt = jnp.sum((keys >= mid).astype(jnp.int32), axis=1, keepdims=True)
        ok = cnt >= K
        return jnp.where(ok, mid, lo), jnp.where(ok, hi, mid)

    lo = jnp.full((BE, 1), MINI32, jnp.int32)
    hi = jnp.full((BE, 1), 2147483647, jnp.int32)
    lo, hi = lax.fori_loop(0, 32, step, (lo, hi))
    thr_ref[...] = jnp.broadcast_to(lo, (BE, 128))


def _thresholds(lg2):
    return pl.pallas_call(
        _thresh_body,
        out_shape=jax.ShapeDtypeStruct((BE, 128), jnp.int32),
    )(lg2)


_GCH = 64


def _route_body(lg2, thr, X2, idx_o, gval_o, xg_o, logv, thrv, idxv, valv, rowv, rows, sem):
    wid = lax.axis_index("s") * NC + lax.axis_index("c")

    @pl.when(wid < BE)
    def _():
        be = wid
        b = be // E
        pltpu.sync_copy(lg2.at[be], logv)
        pltpu.sync_copy(thr.at[be], thrv)
        tvec = thrv[pl.ds(0, L)]
        lane = lax.iota(jnp.int32, L)

        def step(j, off):
            v = logv[pl.ds(j * L, L)]
            m = _f32_key(v) >= tvec
            mi = jnp.where(m, 1, 0)
            pos = jnp.maximum(off + plsc.cumsum(mi) - 1, 0)
            m2 = jnp.logical_and(m, pos < K)
            plsc.store_scatter(idxv, [pos], lane + j * L, mask=m2)
            plsc.store_scatter(valv, [pos], v, mask=m2)
            return off + jnp.sum(mi, axis=0)

        lax.fori_loop(0, T // L, step, jnp.int32(0))
        pltpu.sync_copy(idxv.at[pl.ds(0, K)], idx_o.at[be])
        pltpu.sync_copy(valv.at[pl.ds(0, K)], gval_o.at[be])

        def rstep(c, carry):
            rowv[pl.ds(c * L, L)] = idxv[pl.ds(c * L, L)] + b * T
            return carry

        lax.fori_loop(0, K // L, rstep, 0)
        for c in range(K // _GCH):
            pltpu.async_copy(X2.at[rowv.at[pl.ds(c * _GCH, _GCH)]], rows, sem).wait()
            pltpu.sync_copy(rows, xg_o.at[pl.ds(be * K + c * _GCH, _GCH)])


def _route(lg2, thr, X2):
    f = pl.kernel(
        _route_body,
        out_type=(
            jax.ShapeDtypeStruct((BE, K), jnp.int32),
            jax.ShapeDtypeStruct((BE, K), jnp.float32),
            jax.ShapeDtypeStruct((BE * K, H), jnp.float32),
        ),
        mesh=_sc_mesh(),
        compiler_params=pltpu.CompilerParams(needs_layout_passes=False),
        scratch_types=[
            pltpu.VMEM((T,), jnp.float32),
            pltpu.VMEM((128,), jnp.int32),
            pltpu.VMEM((K + L,), jnp.int32),
            pltpu.VMEM((K + L,), jnp.float32),
            pltpu.VMEM((K,), jnp.int32),
            pltpu.VMEM((_GCH, H), jnp.float32),
            pltpu.SemaphoreType.DMA,
        ],
    )
    return f(lg2, thr, X2)


def _attn_body(xg_ref, wqkv_ref, wo_ref, idx_ref, gval_ref, out_ref):
    xg = xg_ref[0]
    qkv = jnp.dot(xg, wqkv_ref[0], preferred_element_type=jnp.float32)
    q, k, v = qkv[:, :HP], qkv[:, HP:2 * HP], qkv[:, 2 * HP:]

    posr = idx_ref[0].astype(jnp.float32)
    posc = jnp.reshape(posr, (K, 1))
    half = N_ROT // 2
    fr = lax.broadcasted_iota(jnp.int32, (1, half), 1).astype(jnp.float32)
    inv_freq = jnp.exp(fr * (-2.0 * jnp.log(ROPE_BASE) / N_ROT))
    ang = posc * inv_freq
    s, c = jnp.sin(ang), jnp.cos(ang)

    def rope(x):
        x1, x2 = x[:, :half], x[:, half:N_ROT]
        return jnp.concatenate(
            [x1 * c - x2 * s, x1 * s + x2 * c, x[:, N_ROT:]], axis=1)

    qr, kr = rope(q), rope(k)
    scores = lax.dot_general(qr, kr, (((1,), (1,)), ((), ())),
                             preferred_element_type=jnp.float32) * (HP ** -0.5)
    scores = jnp.where(posc >= posr, scores, -1e30)
    mx = jnp.max(scores, axis=1, keepdims=True)
    p = jnp.exp(scores - mx)
    a = p / jnp.sum(p, axis=1, keepdims=True)
    av = jnp.dot(a, v, preferred_element_type=jnp.float32)
    gate = jax.nn.sigmoid(jnp.reshape(gval_ref[0], (K, 1)))
    out_ref[0] = jnp.dot(av * gate, wo_ref[0], preferred_element_type=jnp.float32)


def _attention(xg3, W_qkv, W_o, idx3, gval3):
    return pl.pallas_call(
        _attn_body,
        grid=(BE,),
        in_specs=[
            pl.BlockSpec((1, K, H), lambda i: (i, 0, 0)),
            pl.BlockSpec((1, H, 3 * HP), lambda i: (i % E, 0, 0)),
            pl.BlockSpec((1, HP, H), lambda i: (i % E, 0, 0)),
            pl.BlockSpec((1, 1, K), lambda i: (i, 0, 0)),
            pl.BlockSpec((1, 1, K), lambda i: (i, 0, 0)),
        ],
        out_specs=pl.BlockSpec((1, K, H), lambda i: (i, 0, 0)),
        out_shape=jax.ShapeDtypeStruct((BE, K, H), jnp.float32),
    )(xg3, W_qkv, W_o, idx3, gval3)


_RNG = 128
_NR = T // _RNG
_GC2 = 16


def _combine_body(idx2, Xp2, out1, idxv, dstv, rowv, rows, buf, sem):
    wid = lax.axis_index("s") * NC + lax.axis_index("c")
    lane = lax.iota(jnp.int32, L)

    for rnd in range(B * _NR // NW):
        task = wid * (B * _NR // NW) + rnd
        b = task // _NR
        base = (task % _NR) * _RNG

        def zstep(i, carry):
            buf[pl.ds(i * L, L)] = jnp.zeros((L,), jnp.float32)
            return carry

        lax.fori_loop(0, _RNG * H // L, zstep, 0)

        def rzero(i, carry):
            rowv[pl.ds(i * L, L)] = jnp.zeros((L,), jnp.int32)
            return carry

        lax.fori_loop(0, EK // L, rzero, 0)
        pltpu.sync_copy(idx2.at[b], idxv)

        def fstep(j, nsel):
            iv = idxv[pl.ds(j * L, L)]
            m = jnp.logical_and(iv >= base, iv < base + _RNG)
            mi = jnp.where(m, 1, 0)
            pos = jnp.maximum(nsel + plsc.cumsum(mi) - 1, 0)
            plsc.store_scatter(dstv, [pos], iv - base, mask=m)
            plsc.store_scatter(rowv, [pos], lane + (j * L + b * EK), mask=m)
            return nsel + jnp.sum(mi, axis=0)

        nsel = lax.fori_loop(0, EK // L, fstep, jnp.int32(0))

        def gstep(cidx, carry):
            pltpu.async_copy(Xp2.at[rowv.at[pl.ds(cidx * _GC2, _GC2)]], rows, sem).wait()
            nrow = jnp.minimum(nsel - cidx * _GC2, _GC2)

            def astep(jj, c2):
                dst = dstv[pl.ds(cidx * _GC2 + jj, L)][0]
                for h in range(H // L):
                    sl = pl.ds(dst * H + h * L, L)
                    buf[sl] = buf[sl] + rows[jj, pl.ds(h * L, L)]
                return c2

            lax.fori_loop(0, nrow, astep, 0)
            return carry

        lax.fori_loop(0, (nsel + _GC2 - 1) // _GC2, gstep, 0)
        pltpu.sync_copy(buf, out1.at[pl.ds((b * T + base) * H, _RNG * H)])


def _combine(idx2, Xp2):
    f = pl.kernel(
        _combine_body,
        out_type=jax.ShapeDtypeStruct((B * T * H,), jnp.float32),
        mesh=_sc_mesh(),
        compiler_params=pltpu.CompilerParams(needs_layout_passes=False),
        scratch_types=[
            pltpu.VMEM((EK,), jnp.int32),
            pltpu.VMEM((EK + L,), jnp.int32),
            pltpu.VMEM((EK,), jnp.int32),
            pltpu.VMEM((_GC2, H), jnp.float32),
            pltpu.VMEM((_RNG * H,), jnp.float32),
            pltpu.SemaphoreType.DMA,
        ],
    )
    return f(idx2, Xp2)


def kernel(X, r_w, W_qkv, W_o):
    lg = _router(X, r_w)
    lg2 = jnp.reshape(lg, (BE, T))
    thr = _thresholds(lg2)
    X2 = jnp.reshape(X, (B * T, H))
    idx, gval, xg = _route(lg2, thr, X2)
    xpre = _attention(
        jnp.reshape(xg, (BE, K, H)), W_qkv, W_o,
        jnp.reshape(idx, (BE, 1, K)), jnp.reshape(gval, (BE, 1, K)))
    out1 = _combine(jnp.reshape(idx, (B, EK)), jnp.reshape(xpre, (BE * K, H)))
    return jnp.reshape(out1, (B, T, H))


# speedup vs baseline: 1.5965x; 1.2399x over previous
import functools

import jax
import jax.numpy as jnp
from jax import lax
from jax.experimental import pallas as pl
from jax.experimental.pallas import tpu as pltpu
from jax.experimental.pallas import tpu_sc as plsc

B, T, H = 2, 8192, 768
E, HP = 12, 64
K = 256
N_ROT = 32
ROPE_BASE = 10000.0
BE = B * E
EK = E * K
NC, NS, L = 2, 16, 16
NW = NC * NS
MINI32 = -2147483648

@functools.lru_cache(maxsize=None)
def _sc_mesh():
    return plsc.VectorSubcoreMesh(
        core_axis_name="c", subcore_axis_name="s", num_cores=NC, num_subcores=NS)


def _f32_key(v):
    i = lax.bitcast_convert_type(v, jnp.int32)
    return jnp.where(i >= 0, i, jnp.bitwise_xor(jnp.bitwise_not(i), jnp.int32(MINI32)))


_TCH = 2048


def _router_body(x_ref, w_ref, lg_ref, thr_ref, acc_ref):
    b = pl.program_id(0)
    g = pl.program_id(1)
    x = x_ref[0]
    w = w_ref[...]
    lg = lax.dot_general(w, x, (((1,), (1,)), ((), ())),
                         preferred_element_type=jnp.float32)
    lg_ref[0] = lg
    acc_ref[b, :, pl.ds(g * _TCH, _TCH)] = lg

    @pl.when(jnp.logical_and(b == B - 1, g == T // _TCH - 1))
    def _():
        keys = _f32_key(jnp.reshape(acc_ref[...], (BE, T)))

        def step(_, carry):
            lo, hi = carry
            mid = (lo >> 1) + (hi >> 1) + (lo & hi & 1)
            cnt = jnp.sum((keys >= mid).astype(jnp.int32), axis=1, keepdims=True)
            ok = cnt >= K
            return jnp.where(ok, mid, lo), jnp.where(ok, hi, mid)

        lo = jnp.full((BE, 1), MINI32, jnp.int32)
        hi = jnp.full((BE, 1), 2147483647, jnp.int32)
        lo, hi = lax.fori_loop(0, 32, step, (lo, hi))
        thr_ref[...] = jnp.broadcast_to(lo, (BE, 128))


def _router(X, r_w):
    return pl.pallas_call(
        _router_body,
        grid=(B, T // _TCH),
        in_specs=[
            pl.BlockSpec((1, _TCH, H), lambda b, g: (b, g, 0)),
            pl.BlockSpec((E, H), lambda b, g: (0, 0)),
        ],
        out_specs=[
            pl.BlockSpec((1, E, _TCH), lambda b, g: (b, 0, g)),
            pl.BlockSpec((BE, 128), lambda b, g: (0, 0)),
        ],
        out_shape=[
            jax.ShapeDtypeStruct((B, E, T), jnp.float32),
            jax.ShapeDtypeStruct((BE, 128), jnp.int32),
        ],
        scratch_shapes=[pltpu.VMEM((B, E, T), jnp.float32)],
    )(X, r_w)


_GCH = 64


def _route_body(lg2, thr, X2, idx_o, gval_o, xg_o, logv, thrv, idxv, valv, rowv, rows, sem):
    wid = lax.axis_index("s") * NC + lax.axis_index("c")

    @pl.when(wid < BE)
    def _():
        be = wid
        b = be // E
        pltpu.sync_copy(lg2.at[be], logv)
        pltpu.sync_copy(thr.at[be], thrv)
        tvec = thrv[pl.ds(0, L)]
        lane = lax.iota(jnp.int32, L)

        def step(j, off):
            v = logv[pl.ds(j * L, L)]
            m = _f32_key(v) >= tvec
            mi = jnp.where(m, 1, 0)
            pos = jnp.maximum(off + plsc.cumsum(mi) - 1, 0)
            m2 = jnp.logical_and(m, pos < K)
            plsc.store_scatter(idxv, [pos], lane + j * L, mask=m2)
            plsc.store_scatter(valv, [pos], v, mask=m2)
            return off + plsc.all_reduce_population_count(m)[0]

        lax.fori_loop(0, T // L, step, jnp.int32(0))
        pltpu.sync_copy(idxv.at[pl.ds(0, K)], idx_o.at[be])
        pltpu.sync_copy(valv.at[pl.ds(0, K)], gval_o.at[be])

        def rstep(c, carry):
            rowv[pl.ds(c * L, L)] = idxv[pl.ds(c * L, L)] + b * T
            return carry

        lax.fori_loop(0, K // L, rstep, 0)
        for c in range(K // _GCH):
            pltpu.async_copy(X2.at[rowv.at[pl.ds(c * _GCH, _GCH)]], rows, sem).wait()
            pltpu.sync_copy(rows, xg_o.at[pl.ds(be * K + c * _GCH, _GCH)])


def _route(lg2, thr, X2):
    f = pl.kernel(
        _route_body,
        out_type=(
            jax.ShapeDtypeStruct((BE, K), jnp.int32),
            jax.ShapeDtypeStruct((BE, K), jnp.float32),
            jax.ShapeDtypeStruct((BE * K, H), jnp.float32),
        ),
        mesh=_sc_mesh(),
        compiler_params=pltpu.CompilerParams(needs_layout_passes=False),
        scratch_types=[
            pltpu.VMEM((T,), jnp.float32),
            pltpu.VMEM((128,), jnp.int32),
            pltpu.VMEM((K + L,), jnp.int32),
            pltpu.VMEM((K + L,), jnp.float32),
            pltpu.VMEM((K,), jnp.int32),
            pltpu.VMEM((_GCH, H), jnp.float32),
            pltpu.SemaphoreType.DMA,
        ],
    )
    return f(lg2, thr, X2)


def _attn_body(xg_ref, wqkv_ref, wo_ref, idx_ref, gval_ref, out_ref):
    xg = xg_ref[0]
    qkv = jnp.dot(xg, wqkv_ref[0], preferred_element_type=jnp.float32)
    q, k, v = qkv[:, :HP], qkv[:, HP:2 * HP], qkv[:, 2 * HP:]

    posr = idx_ref[0].astype(jnp.float32)
    posc = jnp.reshape(posr, (K, 1))
    half = N_ROT // 2
    fr = lax.broadcasted_iota(jnp.int32, (1, half), 1).astype(jnp.float32)
    inv_freq = jnp.exp(fr * (-2.0 * jnp.log(ROPE_BASE) / N_ROT))
    ang = posc * inv_freq
    s, c = jnp.sin(ang), jnp.cos(ang)

    def rope(x):
        x1, x2 = x[:, :half], x[:, half:N_ROT]
        return jnp.concatenate(
            [x1 * c - x2 * s, x1 * s + x2 * c, x[:, N_ROT:]], axis=1)

    qr, kr = rope(q), rope(k)
    scores = lax.dot_general(qr, kr, (((1,), (1,)), ((), ())),
                             preferred_element_type=jnp.float32) * (HP ** -0.5)
    scores = jnp.where(posc >= posr, scores, -1e30)
    mx = jnp.max(scores, axis=1, keepdims=True)
    p = jnp.exp(scores - mx)
    a = p / jnp.sum(p, axis=1, keepdims=True)
    av = jnp.dot(a, v, preferred_element_type=jnp.float32)
    gate = jax.nn.sigmoid(jnp.reshape(gval_ref[0], (K, 1)))
    out_ref[0] = jnp.dot(av * gate, wo_ref[0], preferred_element_type=jnp.float32)


def _attention(xg3, W_qkv, W_o, idx3, gval3):
    return pl.pallas_call(
        _attn_body,
        grid=(BE,),
        in_specs=[
            pl.BlockSpec((1, K, H), lambda i: (i, 0, 0)),
            pl.BlockSpec((1, H, 3 * HP), lambda i: (i % E, 0, 0)),
            pl.BlockSpec((1, HP, H), lambda i: (i % E, 0, 0)),
            pl.BlockSpec((1, 1, K), lambda i: (i, 0, 0)),
            pl.BlockSpec((1, 1, K), lambda i: (i, 0, 0)),
        ],
        out_specs=pl.BlockSpec((1, K, H), lambda i: (i, 0, 0)),
        out_shape=jax.ShapeDtypeStruct((BE, K, H), jnp.float32),
    )(xg3, W_qkv, W_o, idx3, gval3)


_RNG = 128
_NR = T // _RNG
_GC2 = 16


def _combine_body(idx2, Xp2, zrow, out1, idxv, dstv, rowv, rows, buf, sem, zsem):
    wid = lax.axis_index("s") * NC + lax.axis_index("c")
    lane = lax.iota(jnp.int32, L)

    for rnd in range(B * _NR // NW):
        task = wid * (B * _NR // NW) + rnd
        b = task // _NR
        base = (task % _NR) * _RNG

        zcp = pltpu.async_copy(zrow, buf, zsem)
        pltpu.sync_copy(idx2.at[b], idxv)

        def fstep(j, nsel):
            iv = idxv[pl.ds(j * L, L)]
            m = jnp.logical_and(iv >= base, iv < base + _RNG)
            mi = jnp.where(m, 1, 0)
            pos = jnp.maximum(nsel + plsc.cumsum(mi) - 1, 0)
            plsc.store_scatter(dstv, [pos], iv - base, mask=m)
            plsc.store_scatter(rowv, [pos], lane + (j * L + b * EK), mask=m)
            return nsel + plsc.all_reduce_population_count(m)[0]

        nsel = lax.fori_loop(0, EK // L, fstep, jnp.int32(0))
        rowv[pl.ds(nsel, L)] = jnp.zeros((L,), jnp.int32)
        zcp.wait()

        def gstep(cidx, carry):
            pltpu.async_copy(Xp2.at[rowv.at[pl.ds(cidx * _GC2, _GC2)]], rows, sem).wait()
            nrow = jnp.minimum(nsel - cidx * _GC2, _GC2)

            def astep(jj, c2):
                dst = dstv[pl.ds(cidx * _GC2 + jj, L)][0]
                for h in range(H // L):
                    sl = pl.ds(dst * H + h * L, L)
                    buf[sl] = buf[sl] + rows[jj, pl.ds(h * L, L)]
                return c2

            lax.fori_loop(0, nrow, astep, 0)
            return carry

        lax.fori_loop(0, (nsel + _GC2 - 1) // _GC2, gstep, 0)
        pltpu.sync_copy(buf, out1.at[pl.ds((b * T + base) * H, _RNG * H)])


def _combine(idx2, Xp2):
    f = pl.kernel(
        _combine_body,
        out_type=jax.ShapeDtypeStruct((B * T * H,), jnp.float32),
        mesh=_sc_mesh(),
        compiler_params=pltpu.CompilerParams(needs_layout_passes=False),
        scratch_types=[
            pltpu.VMEM((EK,), jnp.int32),
            pltpu.VMEM((EK + L,), jnp.int32),
            pltpu.VMEM((EK + L,), jnp.int32),
            pltpu.VMEM((_GC2, H), jnp.float32),
            pltpu.VMEM((_RNG * H,), jnp.float32),
            pltpu.SemaphoreType.DMA,
            pltpu.SemaphoreType.DMA,
        ],
    )
    return f(idx2, Xp2, jnp.zeros((_RNG * H,), jnp.float32))


def kernel(X, r_w, W_qkv, W_o):
    lg, thr = _router(X, r_w)
    lg2 = jnp.reshape(lg, (BE, T))
    X2 = jnp.reshape(X, (B * T, H))
    idx, gval, xg = _route(lg2, thr, X2)
    xpre = _attention(
        jnp.reshape(xg, (BE, K, H)), W_qkv, W_o,
        jnp.reshape(idx, (BE, 1, K)), jnp.reshape(gval, (BE, 1, K)))
    out1 = _combine(jnp.reshape(idx, (B, EK)), jnp.reshape(xpre, (BE * K, H)))
    return jnp.reshape(out1, (B, T, H))
